# Initial kernel scaffold; baseline (speedup 1.0000x reference)
import jax
import jax.numpy as jnp
from jax.experimental import pallas as pl


def kernel(x, edge_index, edge_attr, W1, b1, W2, b2, We, be, Wg, att_src, att_dst, bg):
    raise NotImplementedError("write your pallas kernel here")


# speedup vs baseline: 25.9233x; 25.9233x over previous
import jax
import jax.numpy as jnp
from jax import lax
from jax.experimental import pallas as pl
from jax.experimental.pallas import tpu as pltpu
from jax.experimental.pallas import tpu_sc as plsc

N = 10000
E = 160000
F32 = jnp.float32
I32 = jnp.int32

NPAD = 10240
EPAD = 163840
EXT = 172032
TCH = EXT // 16
ACH = 128
ANC = TCH // ACH
WGRP = 4
NSL = NPAD // 16
ECH = EPAD // 32
GXRP = EPAD // 32
GXK = 128
ABR = NPAD // 32
XPAD = 3 * NPAD

NEG_INF = float("-inf")


def _sc_kernel(out_type, scratch_types):
    def deco(body):
        cache = {}

        def call(*args):
            if "k" not in cache:
                mesh = plsc.VectorSubcoreMesh(
                    core_axis_name="c", subcore_axis_name="s")
                cache["k"] = pl.kernel(
                    body, out_type=out_type, mesh=mesh,
                    compiler_params=pltpu.CompilerParams(
                        needs_layout_passes=False),
                    scratch_types=scratch_types)
            return cache["k"](*args)

        return call

    return deco


def _lrelu(x):
    return jnp.maximum(x, 0.2 * x)


def _finz(v):
    return jnp.where(v == NEG_INF, 0.0, v)


@_sc_kernel(
    out_type=jax.ShapeDtypeStruct((6 * EPAD,), F32),
    scratch_types=[
        pltpu.VMEM((XPAD,), F32),
        pltpu.VMEM((ECH,), I32),
        pltpu.VMEM((ECH,), I32),
        pltpu.VMEM((6, ECH), F32),
    ],
)
def _pk_edge_gather(xf, rowf, colf, gt, xv, rv, cv, gst):
    c = lax.axis_index("c")
    s = lax.axis_index("s")
    base = (s * 2 + c) * ECH
    pltpu.sync_copy(xf, xv)
    pltpu.sync_copy(rowf.at[pl.ds(base, ECH)], rv)
    pltpu.sync_copy(colf.at[pl.ds(base, ECH)], cv)

    def body(v, carry):
        sl = pl.ds(v * 16, 16)
        r16 = rv[sl]
        c16 = cv[sl]
        ci = c16 * 3
        ri = r16 * 3
        for f in range(3):
            xi = plsc.load_gather(xv, [ci + f])
            xj = plsc.load_gather(xv, [ri + f])
            gst[f, sl] = xi
            gst[3 + f, sl] = xj - xi
        return carry

    lax.fori_loop(0, ECH // 16, body, 0)
    for f in range(6):
        pltpu.sync_copy(gst.at[f], gt.at[pl.ds(f * EPAD + base, ECH)])


@_sc_kernel(
    out_type=jax.ShapeDtypeStruct((64 * NPAD,), F32),
    scratch_types=[
        pltpu.VMEM((NPAD,), F32),
        pltpu.VMEM((NPAD,), F32),
        pltpu.VMEM((ECH,), I32),
        pltpu.VMEM((ECH,), F32),
        pltpu.VMEM((ECH,), F32),
    ],
)
def _pk_segmax(ht, colf, xnt, acc0, acc1, cv, h0v, h1v):
    c = lax.axis_index("c")
    s = lax.axis_index("s")
    f0 = (s * 2 + c) * 2
    ninf = jnp.full((16,), NEG_INF, F32)

    def init(v, carry):
        sl = pl.ds(v * 16, 16)
        acc0[sl] = ninf
        acc1[sl] = ninf
        return carry

    lax.fori_loop(0, NPAD // 16, init, 0)

    def chunk(ch, carry):
        eb = ch * ECH
        pltpu.sync_copy(colf.at[pl.ds(eb, ECH)], cv)
        pltpu.sync_copy(ht.at[pl.ds(f0 * EPAD + eb, ECH)], h0v)
        pltpu.sync_copy(ht.at[pl.ds((f0 + 1) * EPAD + eb, ECH)], h1v)

        def vec(v, carry2):
            sl = pl.ds(v * 16, 16)
            c16 = cv[sl]
            h0 = h0v[sl]
            h1 = h1v[sl]
            g0 = plsc.load_gather(acc0, [c16])
            g1 = plsc.load_gather(acc1, [c16])
            plsc.store_scatter(acc0, [c16], h0, mask=h0 > g0)
            plsc.store_scatter(acc1, [c16], h1, mask=h1 > g1)

            def cond(_):
                q0 = plsc.load_gather(acc0, [c16])
                q1 = plsc.load_gather(acc1, [c16])
                bad = jnp.logical_or(h0 > q0, h1 > q1)
                return lax.reduce_max(bad.astype(I32), axes=(0,)) > 0

            def fix(n):
                q0 = plsc.load_gather(acc0, [c16])
                q1 = plsc.load_gather(acc1, [c16])
                plsc.store_scatter(acc0, [c16], h0, mask=h0 > q0)
                plsc.store_scatter(acc1, [c16], h1, mask=h1 > q1)
                return n + 1

            lax.while_loop(cond, fix, 0)
            return carry2

        lax.fori_loop(0, ECH // 16, vec, 0)
        return carry

    lax.fori_loop(0, EPAD // ECH, chunk, 0)

    def fin(v, carry):
        sl = pl.ds(v * 16, 16)
        acc0[sl] = _finz(acc0[sl])
        acc1[sl] = _finz(acc1[sl])
        return carry

    lax.fori_loop(0, NPAD // 16, fin, 0)
    pltpu.sync_copy(acc0, xnt.at[pl.ds(f0 * NPAD, NPAD)])
    pltpu.sync_copy(acc1, xnt.at[pl.ds((f0 + 1) * NPAD, NPAD)])


@_sc_kernel(
    out_type=jax.ShapeDtypeStruct((EPAD, 256), F32),
    scratch_types=[
        pltpu.VMEM((GXRP,), I32),
        pltpu.VMEM((GXK, 256), F32),
        pltpu.VMEM((GXK, 256), F32),
        pltpu.SemaphoreType.DMA,
        pltpu.SemaphoreType.DMA,
        pltpu.SemaphoreType.DMA,
        pltpu.SemaphoreType.DMA,
    ],
)
def _pk_gather_xn2(xn2, rowf, xn2g, rv, mb0, mb1, sg0, sg1, sw0, sw1):
    c = lax.axis_index("c")
    s = lax.axis_index("s")
    base = (s * 2 + c) * GXRP
    pltpu.sync_copy(rowf.at[pl.ds(base, GXRP)], rv)
    n = GXRP // GXK
    mbs = (mb0, mb1)
    sgs = (sg0, sg1)
    sws = (sw0, sw1)

    def fire(i):
        return pltpu.async_copy(
            xn2.at[rv.at[pl.ds(i * GXK, GXK)]], mbs[i % 2], sgs[i % 2])

    d = fire(0)
    wbs = [None, None]
    for i in range(n):
        d.wait()
        if i + 1 < n:
            if wbs[(i + 1) % 2] is not None:
                wbs[(i + 1) % 2].wait()
            d = fire(i + 1)
        wbs[i % 2] = pltpu.async_copy(
            mbs[i % 2], xn2g.at[pl.ds(base + i * GXK, GXK)], sws[i % 2])
    for wb in wbs:
        if wb is not None:
            wb.wait()


@_sc_kernel(
    out_type=jax.ShapeDtypeStruct((32 * 8 * ABR,), F32),
    scratch_types=[
        pltpu.VMEM((8 * NPAD,), F32),
        pltpu.VMEM((8 * ABR,), F32),
        pltpu.VMEM((NPAD,), I32),
        pltpu.VMEM((8 * ABR,), F32),
    ],
)
def _pk_abt(p8f, q8f, rowpad, abt_raw, pv, qb, rowv, tb):
    c = lax.axis_index("c")
    s = lax.axis_index("s")
    t = s * 2 + c
    base = t * ABR
    pltpu.sync_copy(p8f, pv)
    pltpu.sync_copy(q8f.at[pl.ds(base * 8, 8 * ABR)], qb)
    pltpu.sync_copy(rowpad, rowv)
    it = lax.iota(I32, 16)

    def vec(v, carry):
        j16 = it + v * 16
        r16 = rowv[pl.ds(base + v * 16, 16)]
        for k in range(8):
            pvk = plsc.load_gather(pv, [r16 * 8 + k])
            qvk = plsc.load_gather(qb, [j16 * 8 + k])
            tb[pl.ds(k * ABR + v * 16, 16)] = pvk + qvk
        return carry

    lax.fori_loop(0, ABR // 16, vec, 0)
    for k in range(8):
        pltpu.sync_copy(tb.at[pl.ds(k * ABR, ABR)],
                        abt_raw.at[pl.ds(k * NPAD + base, ABR)])


@_sc_kernel(
    out_type=(
        jax.ShapeDtypeStruct((4 * EXT,), F32),
        jax.ShapeDtypeStruct((2 * 2 * NPAD,), F32),
    ),
    scratch_types=[
        pltpu.VMEM((TCH,), I32),
        pltpu.VMEM((TCH,), I32),
        pltpu.VMEM((NPAD,), F32),
        pltpu.VMEM((NPAD,), F32),
        pltpu.VMEM((NPAD,), F32),
        pltpu.VMEM((NPAD,), F32),
        pltpu.VMEM((NPAD,), F32),
        pltpu.VMEM((NPAD,), F32),
        pltpu.VMEM((NPAD,), F32),
        pltpu.VMEM((NPAD,), F32),
        pltpu.VMEM((16, 128), F32),
        pltpu.VMEM((128,), F32),
        pltpu.VMEM((2, WGRP * ACH), F32),
        pltpu.SemaphoreType.DMA,
        pltpu.VMEM_SHARED((2, 16, NPAD), F32),
        pltpu.VMEM_SHARED((2, NPAD), F32),
    ],
)
def _pk_att12(abt, rext, cext, w_hbm, rd_hbm, ridx, cidx, a0, a1, b0, b1,
              m0, m1, acc0, acc1, tb, mbuf, wb, csem, cmb, mden):
    c = lax.axis_index("c")
    s = lax.axis_index("s")
    myc = s * NSL

    pltpu.sync_copy(rext.at[pl.ds(s * TCH, TCH)], ridx)
    pltpu.sync_copy(cext.at[pl.ds(s * TCH, TCH)], cidx)
    pltpu.sync_copy(abt.at[pl.ds((2 * c) * NPAD, NPAD)], a0)
    pltpu.sync_copy(abt.at[pl.ds((2 * c + 1) * NPAD, NPAD)], a1)
    pltpu.sync_copy(abt.at[pl.ds((4 + 2 * c) * NPAD, NPAD)], b0)
    pltpu.sync_copy(abt.at[pl.ds((5 + 2 * c) * NPAD, NPAD)], b1)

    ninf = jnp.full((16,), NEG_INF, F32)

    def init(v, carry):
        sl = pl.ds(v * 16, 16)
        acc0[sl] = ninf
        acc1[sl] = ninf
        return carry

    lax.fori_loop(0, NPAD // 16, init, 0)

    def p1vec(v, carry):
        sl = pl.ds(v * 16, 16)
        r16 = ridx[sl]
        c16 = cidx[sl]
        a0v = plsc.load_gather(a0, [r16])
        a1v = plsc.load_gather(a1, [r16])
        g0 = plsc.load_gather(acc0, [c16])
        g1 = plsc.load_gather(acc1, [c16])
        plsc.store_scatter(acc0, [c16], a0v, mask=a0v > g0)
        plsc.store_scatter(acc1, [c16], a1v, mask=a1v > g1)

        def cond(_):
            q0 = plsc.load_gather(acc0, [c16])
            q1 = plsc.load_gather(acc1, [c16])
            bad = jnp.logical_or(a0v > q0, a1v > q1)
            return lax.reduce_max(bad.astype(I32), axes=(0,)) > 0

        def fix(n):
            q0 = plsc.load_gather(acc0, [c16])
            q1 = plsc.load_gather(acc1, [c16])
            plsc.store_scatter(acc0, [c16], a0v, mask=a0v > q0)
            plsc.store_scatter(acc1, [c16], a1v, mask=a1v > q1)
            return n + 1

        lax.while_loop(cond, fix, 0)
        return carry

    lax.fori_loop(0, TCH // 16, p1vec, 0)

    pltpu.sync_copy(acc0, cmb.at[0, s])
    pltpu.sync_copy(acc1, cmb.at[1, s])
    plsc.subcore_barrier()
    for h in range(2):
        bt = (b0, b1)[h]
        for blk in range(NSL // 128):
            off = myc + blk * 128
            descs = [pltpu.async_copy(
                cmb.at[h, t, pl.ds(off, 128)], tb.at[t], csem)
                for t in range(16)]
            for dsc in descs:
                dsc.wait()

            def red(v, carry, _off=off):
                sl = pl.ds(v * 16, 16)
                m = tb[0, sl]
                for t in range(1, 16):
                    m = jnp.maximum(m, tb[t, sl])
                bv = bt[pl.ds(_off + v * 16, 16)]
                mbuf[sl] = _lrelu(m + bv)
                return carry

            lax.fori_loop(0, 128 // 16, red, 0)
            pltpu.sync_copy(mbuf, mden.at[h, pl.ds(off, 128)])
    plsc.subcore_barrier()
    pltpu.sync_copy(mden.at[0], m0)
    pltpu.sync_copy(mden.at[1], m1)

    zero = jnp.zeros((16,), F32)

    def zinit(v, carry):
        sl = pl.ds(v * 16, 16)
        acc0[sl] = zero
        acc1[sl] = zero
        return carry

    lax.fori_loop(0, NPAD // 16, zinit, 0)

    def p2group(g, carry):
        def p2vec(k, carry2):
            sl = pl.ds(g * (WGRP * ACH) + k * 16, 16)
            r16 = ridx[sl]
            c16 = cidx[sl]
            a0v = plsc.load_gather(a0, [r16])
            a1v = plsc.load_gather(a1, [r16])
            b0v = plsc.load_gather(b0, [c16])
            b1v = plsc.load_gather(b1, [c16])
            m0v = plsc.load_gather(m0, [c16])
            m1v = plsc.load_gather(m1, [c16])
            w0v = jnp.exp(_lrelu(a0v + b0v) - m0v)
            w1v = jnp.exp(_lrelu(a1v + b1v) - m1v)
            wsl = pl.ds(k * 16, 16)
            wb[0, wsl] = w0v
            wb[1, wsl] = w1v
            plsc.addupdate_scatter(acc0, [c16], w0v)
            plsc.addupdate_scatter(acc1, [c16], w1v)
            return carry2

        lax.fori_loop(0, (WGRP * ACH) // 16, p2vec, 0)
        wbase = s * TCH + g * (WGRP * ACH)
        pltpu.sync_copy(
            wb.at[0], w_hbm.at[pl.ds((2 * c) * EXT + wbase, WGRP * ACH)])
        pltpu.sync_copy(
            wb.at[1], w_hbm.at[pl.ds((2 * c + 1) * EXT + wbase, WGRP * ACH)])
        return carry

    lax.fori_loop(0, ANC // WGRP, p2group, 0)

    pltpu.sync_copy(acc0, cmb.at[0, s])
    pltpu.sync_copy(acc1, cmb.at[1, s])
    plsc.subcore_barrier()
    for h in range(2):
        for blk in range(NSL // 128):
            off = myc + blk * 128
            descs = [pltpu.async_copy(
                cmb.at[h, t, pl.ds(off, 128)], tb.at[t], csem)
                for t in range(16)]
            for dsc in descs:
                dsc.wait()

            def redsum(v, carry):
                sl = pl.ds(v * 16, 16)
                m = tb[0, sl]
                for t in range(1, 16):
                    m = m + tb[t, sl]
                mbuf[sl] = 1.0 / (m + 1e-16)
                return carry

            lax.fori_loop(0, 128 // 16, redsum, 0)
            pltpu.sync_copy(
                mbuf, rd_hbm.at[pl.ds((c * 2 + h) * NPAD + off, 128)])


BCH = 64
BNC = TCH // BCH
BWG = 8
BNG = BNC // BWG


@_sc_kernel(
    out_type=jax.ShapeDtypeStruct((2, NPAD, 128), F32),
    scratch_types=[
        pltpu.VMEM((BWG, BCH), I32),
        pltpu.VMEM((BWG, BCH), I32),
        pltpu.VMEM((BWG * BCH,), F32),
        pltpu.VMEM((BWG * BCH,), F32),
        pltpu.VMEM((BCH, 128), F32),
        pltpu.VMEM((BCH, 128), F32),
        pltpu.VMEM((16, 128), F32),
        pltpu.SemaphoreType.DMA,
        pltpu.SemaphoreType.DMA,
        pltpu.VMEM_SHARED((NPAD, 128), F32),
    ],
)
def _pk_att3(rextp, cextp, w_hbm, s2, num3, rgb, cgb, wg0, wg1,
             mb0, mb1, zb, sg0, sg1, numacc):
    c = lax.axis_index("c")
    s = lax.axis_index("s")
    myc = s * NSL

    zero = jnp.zeros((16,), F32)
    for r in range(16):
        for q in range(8):
            zb[r, pl.ds(q * 16, 16)] = zero

    def zrow(i, carry):
        pltpu.sync_copy(zb, numacc.at[pl.ds(myc + i * 16, 16)])
        return carry

    lax.fori_loop(0, NSL // 16, zrow, 0)
    plsc.subcore_barrier()

    mbs = (mb0, mb1)
    sgs = (sg0, sg1)
    wb0 = (2 * c) * EXT + s * TCH
    wb1 = (2 * c + 1) * EXT + s * TCH
    rrow0 = c * (EXT // BCH) + s * BNC
    crow0 = s * BNC

    def group(g, carry):
        pltpu.sync_copy(rextp.at[pl.ds(rrow0 + g * BWG, BWG)], rgb)
        pltpu.sync_copy(cextp.at[pl.ds(crow0 + g * BWG, BWG)], cgb)
        pltpu.sync_copy(
            w_hbm.at[pl.ds(wb0 + g * (BWG * BCH), BWG * BCH)], wg0)
        pltpu.sync_copy(
            w_hbm.at[pl.ds(wb1 + g * (BWG * BCH), BWG * BCH)], wg1)

        d = pltpu.async_copy(s2.at[rgb.at[0]], mbs[0], sgs[0])
        for j in range(BWG):
            which = j % 2
            d.wait()
            if j + 1 < BWG:
                d = pltpu.async_copy(
                    s2.at[rgb.at[j + 1]], mbs[(j + 1) % 2], sgs[(j + 1) % 2])
            mb = mbs[which]
            woff = j * BCH

            def witem(i, carry2, _woff=woff, _mb=mb):
                i16 = jnp.full((16,), _woff + i, I32)
                w0s = plsc.load_gather(wg0, [i16])
                w1s = plsc.load_gather(wg1, [i16])
                for q in range(4):
                    sl = pl.ds(q * 16, 16)
                    _mb[i, sl] = _mb[i, sl] * w0s
                for q in range(4):
                    sl = pl.ds(64 + q * 16, 16)
                    _mb[i, sl] = _mb[i, sl] * w1s
                return carry2

            lax.fori_loop(0, BCH, witem, 0)
            pltpu.sync_copy(mb, numacc.at[cgb.at[j]], add=True)
        return carry

    lax.fori_loop(0, BNG, group, 0)

    plsc.subcore_barrier()
    pltpu.sync_copy(numacc.at[pl.ds(myc, NSL)], num3.at[c, pl.ds(myc, NSL)])


def _tc1_body(g_ref, w1_ref, b1_ref, w2_ref, b2_ref, h_ref):
    t = lax.dot_general(w1_ref[...], g_ref[...], (((0,), (0,)), ((), ())),
                        preferred_element_type=F32)
    t = jnp.maximum(t + b1_ref[...], 0.0)
    h = lax.dot_general(w2_ref[...], t, (((0,), (0,)), ((), ())),
                        preferred_element_type=F32)
    h_ref[...] = h + b2_ref[...]


def _tc2_body(xnt_ref, wga_ref, ea_ref, wep_ref, be_ref, wgb_ref,
              asd_ref, xn2_ref, p8_ref, q8_ref, xe2_ref):
    xnb = lax.dot_general(xnt_ref[...], wga_ref[...], (((0,), (0,)), ((), ())),
                          preferred_element_type=F32)
    xn2_ref[...] = xnb
    p8_ref[...] = jnp.dot(xnb, asd_ref[...], preferred_element_type=F32)
    eh = jnp.maximum(jnp.dot(ea_ref[...], wep_ref[...],
                             preferred_element_type=F32) + be_ref[...], 0.0)
    xeb = jnp.dot(eh, wgb_ref[...], preferred_element_type=F32)
    xe2_ref[...] = xeb
    q8_ref[...] = jnp.dot(xeb, asd_ref[...], preferred_element_type=F32)


def _tcs_body(xn2g_ref, xe2_ref, s2_ref):
    s2_ref[...] = xn2g_ref[...] + xe2_ref[...]


def _tc3_body(xn2g_ref, ea_ref, wep_ref, be_ref, wgb_ref, bg_ref, out_ref):
    eh = jnp.maximum(jnp.dot(ea_ref[...], wep_ref[...],
                             preferred_element_type=F32) + be_ref[...], 0.0)
    out_ref[...] = (xn2g_ref[...]
                    + jnp.dot(eh, wgb_ref[...], preferred_element_type=F32)
                    + bg_ref[...])


def _tcf_body(num_ref, rd_ref, bg_ref, alias_ref, out_ref):
    nb = num_ref[...]
    rb = rd_ref[...]
    bd = rb.shape[0]
    parts = []
    for h2 in range(2):
        rx0 = jnp.broadcast_to(rb[:, 2 * h2][:, None], (bd, 64))
        rx1 = jnp.broadcast_to(rb[:, 2 * h2 + 1][:, None], (bd, 64))
        parts.append(nb[h2] * jnp.concatenate([rx0, rx1], axis=1))
    out_ref[...] = jnp.concatenate(parts, axis=1) + bg_ref[...]


def kernel(x, edge_index, edge_attr, W1, b1, W2, b2, We, be, Wg,
           att_src, att_dst, bg):
    row = edge_index[0].astype(I32)
    col = edge_index[1].astype(I32)

    xf = jnp.concatenate([x.reshape(-1), jnp.zeros((XPAD - 3 * N,), F32)])
    row_pk = jnp.concatenate([row, jnp.zeros((EPAD - E,), I32)])
    col_pk = jnp.concatenate([col, jnp.full((EPAD - E,), NPAD - 1, I32)])
    arN = jnp.arange(N, dtype=I32)
    r_ext = jnp.concatenate([row, arN, jnp.full((EXT - E - N,), NPAD - 1, I32)])
    c_ext = jnp.concatenate([col, arN, jnp.full((EXT - E - N,), NPAD - 1, I32)])
    rowpad = jnp.concatenate([row[:N], jnp.zeros((NPAD - N,), I32)])
    WgA = Wg[:64]
    WgB = Wg[64:]
    We_pad = jnp.concatenate([We, jnp.zeros((6, 64), F32)], axis=0)
    ea_pad = jnp.pad(edge_attr, ((0, 0), (0, 6)))
    ea10k = jnp.pad(edge_attr[:N], ((0, NPAD - N), (0, 6)))
    blk = jnp.kron(jnp.eye(4, dtype=F32), jnp.ones((64, 1), F32))
    asd = jnp.concatenate(
        [blk * att_src.reshape(256, 1), blk * att_dst.reshape(256, 1),
         jnp.zeros((256, 8), F32)], axis=1)
    b1c = b1.reshape(64, 1)
    b2c = b2.reshape(64, 1)
    be2 = be.reshape(1, 64)
    bg2 = bg.reshape(1, 256)

    g1d = _pk_edge_gather(xf, row_pk, col_pk)
    g_T = g1d.reshape(6, EPAD)

    h_T = pl.pallas_call(
        _tc1_body,
        grid=(EPAD // 640,),
        in_specs=[
            pl.BlockSpec((6, 640), lambda i: (0, i)),
            pl.BlockSpec((6, 64), lambda i: (0, 0)),
            pl.BlockSpec((64, 1), lambda i: (0, 0)),
            pl.BlockSpec((64, 64), lambda i: (0, 0)),
            pl.BlockSpec((64, 1), lambda i: (0, 0)),
        ],
        out_specs=pl.BlockSpec((64, 640), lambda i: (0, i)),
        out_shape=jax.ShapeDtypeStruct((64, EPAD), F32),
    )(g_T, W1, b1c, W2, b2c)

    xnt1d = _pk_segmax(h_T.reshape(-1), col_pk)
    xnt = xnt1d.reshape(64, NPAD)

    xn2, P8, Q8, xe2 = pl.pallas_call(
        _tc2_body,
        grid=(NPAD // 1024,),
        in_specs=[
            pl.BlockSpec((64, 1024), lambda i: (0, i)),
            pl.BlockSpec((64, 256), lambda i: (0, 0)),
            pl.BlockSpec((1024, 8), lambda i: (i, 0)),
            pl.BlockSpec((8, 64), lambda i: (0, 0)),
            pl.BlockSpec((1, 64), lambda i: (0, 0)),
            pl.BlockSpec((64, 256), lambda i: (0, 0)),
            pl.BlockSpec((256, 16), lambda i: (0, 0)),
        ],
        out_specs=[
            pl.BlockSpec((1024, 256), lambda i: (i, 0)),
            pl.BlockSpec((1024, 16), lambda i: (i, 0)),
            pl.BlockSpec((1024, 16), lambda i: (i, 0)),
            pl.BlockSpec((1024, 256), lambda i: (i, 0)),
        ],
        out_shape=[
            jax.ShapeDtypeStruct((NPAD, 256), F32),
            jax.ShapeDtypeStruct((NPAD, 16), F32),
            jax.ShapeDtypeStruct((NPAD, 16), F32),
            jax.ShapeDtypeStruct((NPAD, 256), F32),
        ],
    )(xnt, WgA, ea10k, We_pad, be2, WgB, asd)

    xn2g = _pk_gather_xn2(xn2, row_pk)

    s2 = pl.pallas_call(
        _tcs_body,
        grid=(2, NPAD // 1024),
        in_specs=[
            pl.BlockSpec((1024, 128), lambda h, i: (i, h)),
            pl.BlockSpec((1024, 128), lambda h, i: (i, h)),
        ],
        out_specs=pl.BlockSpec(
            (1024, 128), lambda h, i: (h * (NPAD // 1024) + i, 0)),
        out_shape=jax.ShapeDtypeStruct((2 * NPAD, 128), F32),
    )(xn2g, xe2)

    abt = _pk_abt(P8[:, :8].reshape(-1), Q8[:, :8].reshape(-1), rowpad)
    w_hbm, rd1d = _pk_att12(abt, r_ext, c_ext)
    rextp = jnp.concatenate([r_ext, r_ext + NPAD]).reshape(-1, BCH)
    num3 = _pk_att3(rextp, c_ext.reshape(-1, BCH), w_hbm, s2)
    rd4 = rd1d.reshape(4, NPAD).T

    out_base = pl.pallas_call(
        _tc3_body,
        grid=(E // 2000,),
        in_specs=[
            pl.BlockSpec((2000, 256), lambda i: (i, 0)),
            pl.BlockSpec((2000, 8), lambda i: (i, 0)),
            pl.BlockSpec((8, 64), lambda i: (0, 0)),
            pl.BlockSpec((1, 64), lambda i: (0, 0)),
            pl.BlockSpec((64, 256), lambda i: (0, 0)),
            pl.BlockSpec((1, 256), lambda i: (0, 0)),
        ],
        out_specs=pl.BlockSpec((2000, 256), lambda i: (i, 0)),
        out_shape=jax.ShapeDtypeStruct((E, 256), F32),
    )(xn2g, ea_pad, We_pad, be2, WgB, bg2)

    out = pl.pallas_call(
        _tcf_body,
        grid=(N // 2000,),
        in_specs=[
            pl.BlockSpec((2, 2000, 128), lambda i: (0, i, 0)),
            pl.BlockSpec((2000, 4), lambda i: (i, 0)),
            pl.BlockSpec((1, 256), lambda i: (0, 0)),
            pl.BlockSpec(memory_space=pl.ANY),
        ],
        out_specs=pl.BlockSpec((2000, 256), lambda i: (i, 0)),
        out_shape=jax.ShapeDtypeStruct((E, 256), F32),
        input_output_aliases={3: 0},
    )(num3, rd4, bg2, out_base)

    return out
